# baseline (speedup 1.0000x reference)
import functools

import jax
import jax.numpy as jnp
from jax import lax
from jax.experimental import pallas as pl
from jax.experimental.pallas import tpu as pltpu
from jax.experimental.pallas import tpu_sc as plsc

_BATCH, _HIST, _DIM = 16384, 50, 32
_VOCAB = 1000000
_SDIM = 40

_info = plsc.get_sparse_core_info()
_NC, _NS = _info.num_cores, _info.num_subcores
_NW = _NC * _NS
_BPW = _BATCH // _NW
_NJ = _BPW // 128
_SK = 129
_NBB = _BATCH // 128
_HSLAB = (_DIM // 8) * _NBB

_W = 800
_NK = _VOCAB // _W
_NI = (_NK + _NW - 1) // _NW

_mesh = plsc.VectorSubcoreMesh(core_axis_name="c", subcore_axis_name="s")


@functools.partial(
    pl.kernel,
    mesh=_mesh,
    out_type=jax.ShapeDtypeStruct((_VOCAB * _SDIM,), jnp.float32),
    scratch_types=[
        pltpu.VMEM((2, _DIM, _W + 1), jnp.float32),
        pltpu.VMEM((2, _W * _SDIM), jnp.float32),
        pltpu.SemaphoreType.DMA,
        pltpu.SemaphoreType.DMA,
    ],
    compiler_params=pltpu.CompilerParams(
        use_tc_tiling_on_sc=False, needs_layout_passes=False
    ),
)
def _transpose_table(embt_hbm, out_hbm, skbuf, obuf, isem, osem):
    wid = lax.axis_index("s") * _NC + lax.axis_index("c")
    iota16 = lax.iota(jnp.int32, 16)
    row_lo = iota16
    row_hi = iota16 + 16

    def step(i, carry):
        b2 = (i - 1) % 2
        k_new = i * _NW + wid
        k_cur = (i - 1) * _NW + wid

        @pl.when(jnp.logical_and(i < _NI, k_new < _NK))
        def _fire_in():
            pltpu.async_copy(
                embt_hbm.at[:, pl.ds(k_new * _W, _W)],
                skbuf.at[i % 2, :, pl.ds(0, _W)],
                isem,
            )

        @pl.when(jnp.logical_and(i >= 1, k_cur < _NK))
        def _process():
            pltpu.make_async_copy(
                embt_hbm.at[:, pl.ds(0, _W)],
                skbuf.at[b2, :, pl.ds(0, _W)],
                isem,
            ).wait()

            @pl.when((i - 3) * _NW + wid >= 0)
            def _wait_out():
                pltpu.make_async_copy(
                    out_hbm.at[pl.ds(0, _W * _SDIM)], obuf.at[b2], osem
                ).wait()

            def tbody(vc, c2):
                for t in range(16):
                    v = vc * 16 + t
                    col = jnp.full((16,), 0, jnp.int32) + v
                    lo = plsc.load_gather(skbuf.at[b2], [row_lo, col])
                    hi = plsc.load_gather(skbuf.at[b2], [row_hi, col])
                    obuf[b2, pl.ds(v * _SDIM, 16)] = lo
                    obuf[b2, pl.ds(v * _SDIM + 16, 16)] = hi
                return c2

            lax.fori_loop(0, _W // 16, tbody, 0)
            pltpu.async_copy(
                obuf.at[b2],
                out_hbm.at[pl.ds(k_cur * (_W * _SDIM), _W * _SDIM)],
                osem,
            )

        return carry

    lax.fori_loop(0, _NI + 1, step, 0)
    for b in (0, 1):
        pltpu.make_async_copy(
            out_hbm.at[pl.ds(0, _W * _SDIM)], obuf.at[b], osem
        ).wait()


@functools.partial(
    pl.kernel,
    mesh=_mesh,
    out_type=jax.ShapeDtypeStruct((_HIST * _HSLAB, 8, 128), jnp.float32),
    scratch_types=[
        pltpu.VMEM((_HIST, _BPW), jnp.int32),
        pltpu.VMEM((2 * _NJ, 128, _SDIM), jnp.float32),
        pltpu.VMEM((2, _NJ, _DIM, _SK), jnp.float32),
        pltpu.SemaphoreType.DMA,
        pltpu.SemaphoreType.DMA,
    ],
    compiler_params=pltpu.CompilerParams(
        use_tc_tiling_on_sc=False, needs_layout_passes=False
    ),
)
def _emb_gather(xt_hbm, table_hbm, out_hbm, idx_v, gbuf, tbuf, gsem, osem):
    wid = lax.axis_index("s") * _NC + lax.axis_index("c")
    b0 = wid * _BPW
    pltpu.sync_copy(xt_hbm.at[:, pl.ds(b0, _BPW)], idx_v)

    iota16 = lax.iota(jnp.int32, 16)
    c_lo = iota16
    c_hi = iota16 + 16

    def step(h, carry):
        b2 = (h - 1) % 2

        @pl.when(h < _HIST)
        def _fire_gathers():
            for j in range(_NJ):
                pltpu.async_copy(
                    table_hbm.at[idx_v.at[h, pl.ds(j * 128, 128)]],
                    gbuf.at[(h % 2) * _NJ + j],
                    gsem,
                )

        @pl.when(h >= 1)
        def _transpose_and_writeout():
            hh = h - 1
            for j in range(_NJ):
                pltpu.make_async_copy(
                    table_hbm.at[pl.ds(0, 128)],
                    gbuf.at[b2 * _NJ + j],
                    gsem,
                ).wait()

            @pl.when(h >= 3)
            def _wait_writeout():
                for rr in range(_DIM // 8):
                    for j in range(_NJ):
                        pltpu.make_async_copy(
                            tbuf.at[b2, j, pl.ds(rr * 8, 8), pl.ds(0, 128)],
                            out_hbm.at[0],
                            osem,
                        ).wait()

            def tbody(j, c2):
                blk = gbuf.at[b2 * _NJ + j]
                dst = tbuf.at[b2, j]

                def bbody(bb, c3):
                    bsplat = c3 + bb * 16
                    for t in range(16):
                        boff = bb * 16 + t
                        lo = blk.at[boff, pl.ds(0, 16)][...]
                        hi = blk.at[boff, pl.ds(16, 16)][...]
                        plsc.store_scatter(dst, [c_lo, bsplat + t], lo)
                        plsc.store_scatter(dst, [c_hi, bsplat + t], hi)
                    return c3

                lax.fori_loop(0, 8, bbody, jnp.zeros((16,), jnp.int32))
                return c2

            lax.fori_loop(0, _NJ, tbody, 0)

            for rr in range(_DIM // 8):
                def jbody(j, c4):
                    row = hh * _HSLAB + rr * _NBB + wid * _NJ + j
                    pltpu.async_copy(
                        tbuf.at[b2, j, pl.ds(rr * 8, 8), pl.ds(0, 128)],
                        out_hbm.at[row],
                        osem,
                    )
                    return c4
                lax.fori_loop(0, _NJ, jbody, 0)

        return carry

    lax.fori_loop(0, _HIST + 1, step, 0)
    for b2 in (0, 1):
        for rr in range(_DIM // 8):
            for j in range(_NJ):
                pltpu.make_async_copy(
                    tbuf.at[b2, j, pl.ds(rr * 8, 8), pl.ds(0, 128)],
                    out_hbm.at[0],
                    osem,
                ).wait()


def kernel(x, embedding):
    tflat = _transpose_table(embedding.T)
    table = tflat.reshape(_VOCAB, _SDIM)
    out = _emb_gather(x.T, table)
    out = out.reshape(_HIST, _DIM // 8, _BATCH // 128, 8, 128)
    return out.transpose(2, 4, 0, 1, 3).reshape(_BATCH, _HIST, _DIM)


# speedup vs baseline: 4.6549x; 4.6549x over previous
import functools

import jax
import jax.numpy as jnp
from jax import lax
from jax.experimental import pallas as pl
from jax.experimental.pallas import tpu as pltpu
from jax.experimental.pallas import tpu_sc as plsc

_BATCH, _HIST, _DIM = 16384, 50, 32
_VOCAB = 1000000

_info = plsc.get_sparse_core_info()
_NC, _NS = _info.num_cores, _info.num_subcores
_NW = _NC * _NS
_BPW = _BATCH // _NW
_NJ = _BPW // 128
_SK = 129
_NBB = _BATCH // 128
_HSLAB = (_DIM // 8) * _NBB

_mesh = plsc.VectorSubcoreMesh(core_axis_name="c", subcore_axis_name="s")


@functools.partial(
    pl.kernel,
    mesh=_mesh,
    out_type=jax.ShapeDtypeStruct((_HIST * _HSLAB, 8, 128), jnp.float32),
    scratch_types=[
        pltpu.VMEM((_HIST, _BPW), jnp.int32),
        pltpu.VMEM((2 * _NJ, 128, _DIM), jnp.float32),
        pltpu.VMEM((2, _NJ, _DIM, _SK), jnp.float32),
        pltpu.SemaphoreType.DMA,
        pltpu.SemaphoreType.DMA,
    ],
    compiler_params=pltpu.CompilerParams(
        use_tc_tiling_on_sc=False, needs_layout_passes=False
    ),
)
def _emb_gather(xt_hbm, table_hbm, out_hbm, idx_v, gbuf, tbuf, gsem, osem):
    wid = lax.axis_index("s") * _NC + lax.axis_index("c")
    b0 = wid * _BPW
    pltpu.sync_copy(xt_hbm.at[:, pl.ds(b0, _BPW)], idx_v)

    iota16 = lax.iota(jnp.int32, 16)
    c_lo = iota16
    c_hi = iota16 + 16

    def step(h, carry):
        b2 = (h - 1) % 2

        @pl.when(h < _HIST)
        def _fire_gathers():
            for j in range(_NJ):
                pltpu.async_copy(
                    table_hbm.at[idx_v.at[h, pl.ds(j * 128, 128)]],
                    gbuf.at[(h % 2) * _NJ + j],
                    gsem,
                )

        @pl.when(h >= 1)
        def _transpose_and_writeout():
            hh = h - 1
            for j in range(_NJ):
                pltpu.make_async_copy(
                    table_hbm.at[pl.ds(0, 128)],
                    gbuf.at[b2 * _NJ + j],
                    gsem,
                ).wait()

            @pl.when(h >= 3)
            def _wait_writeout():
                for rr in range(_DIM // 8):
                    for j in range(_NJ):
                        pltpu.make_async_copy(
                            tbuf.at[b2, j, pl.ds(rr * 8, 8), pl.ds(0, 128)],
                            out_hbm.at[0],
                            osem,
                        ).wait()

            def tbody(j, c2):
                blk = gbuf.at[b2 * _NJ + j]
                dst = tbuf.at[b2, j]

                def bbody(bb, c3):
                    bsplat = c3 + bb * 16
                    for t in range(16):
                        boff = bb * 16 + t
                        lo = blk.at[boff, pl.ds(0, 16)][...]
                        hi = blk.at[boff, pl.ds(16, 16)][...]
                        bt = bsplat + t
                        plsc.store_scatter(dst, [c_lo, bt], lo)
                        plsc.store_scatter(dst, [c_hi, bt], hi)
                    return c3

                lax.fori_loop(0, 8, bbody, jnp.zeros((16,), jnp.int32))
                return c2

            lax.fori_loop(0, _NJ, tbody, 0)

            for rr in range(_DIM // 8):
                def jbody(j, c4):
                    row = hh * _HSLAB + rr * _NBB + wid * _NJ + j
                    pltpu.async_copy(
                        tbuf.at[b2, j, pl.ds(rr * 8, 8), pl.ds(0, 128)],
                        out_hbm.at[row],
                        osem,
                    )
                    return c4
                lax.fori_loop(0, _NJ, jbody, 0)

        return carry

    lax.fori_loop(0, _HIST + 1, step, 0)
    for b2 in (0, 1):
        for rr in range(_DIM // 8):
            for j in range(_NJ):
                pltpu.make_async_copy(
                    tbuf.at[b2, j, pl.ds(rr * 8, 8), pl.ds(0, 128)],
                    out_hbm.at[0],
                    osem,
                ).wait()


def kernel(x, embedding):
    table = jnp.pad(embedding, ((0, 0), (0, 128 - _DIM))).reshape(
        4 * _VOCAB, _DIM
    )
    out = _emb_gather((x * 4).T, table)
    out = out.reshape(_HIST, _DIM // 8, _BATCH // 128, 8, 128)
    return out.transpose(2, 4, 0, 1, 3).reshape(_BATCH, _HIST, _DIM)
